# baseline (speedup 1.0000x reference)
import functools

import jax
import jax.numpy as jnp
from jax import lax
from jax.experimental import pallas as pl
from jax.experimental.pallas import tpu as pltpu
from jax.experimental.pallas import tpu_sc as plsc

_M = 32
_NCOLS = 33
_LANES = 16
_NUM_CORES = 2
_NUM_SUBCORES = 16
_NUM_WORKERS = _NUM_CORES * _NUM_SUBCORES


def _make_sc_call(batch):
    rows_per_worker = batch // _NUM_WORKERS
    groups = rows_per_worker // _LANES
    mesh = plsc.VectorSubcoreMesh(
        core_axis_name="c", subcore_axis_name="s",
        num_cores=_NUM_CORES, num_subcores=_NUM_SUBCORES)

    @functools.partial(
        pl.kernel,
        mesh=mesh,
        out_type=jax.ShapeDtypeStruct((batch,), jnp.int32),
        scratch_types=[
            pltpu.VMEM((rows_per_worker, _M), jnp.int32),
            pltpu.VMEM((_NCOLS, _NCOLS), jnp.int32),
            pltpu.VMEM((rows_per_worker,), jnp.int32),
        ],
        compiler_params=pltpu.CompilerParams(
            needs_layout_passes=False,
            disable_bounds_checks=True,
            disable_semaphore_checks=True,
            skip_device_barrier=True,
        ),
    )
    def sc_rank(x_hbm, comb_hbm, out_hbm, x_v, comb_v, out_v):
        wid = (lax.axis_index("s") * jnp.int32(_NUM_CORES)
               + lax.axis_index("c"))
        rbase = wid * jnp.int32(rows_per_worker)
        pltpu.sync_copy(comb_hbm, comb_v)
        pltpu.sync_copy(x_hbm.at[pl.ds(rbase, rows_per_worker)], x_v)
        lane = lax.iota(jnp.int32, _LANES)

        def group_body(g, carry):
            rows = g * jnp.int32(_LANES) + lane
            presum = jnp.zeros((_LANES,), jnp.int32)
            acc = jnp.zeros((_LANES,), jnp.int32)
            for i in range(_M):
                col_i = jnp.full((_LANES,), i, jnp.int32)
                xi = plsc.load_gather(x_v, [rows, col_i])
                left = jnp.full((_LANES,), _M, jnp.int32) - presum
                row_i = jnp.full((_LANES,), _M - 1 - i, jnp.int32)
                cval = plsc.load_gather(comb_v, [row_i, left])
                acc = acc + cval * xi
                presum = presum + xi
            out_v[pl.ds(g * jnp.int32(_LANES), _LANES)] = acc
            return carry

        lax.fori_loop(jnp.int32(0), jnp.int32(groups), group_body,
                      jnp.int32(0))
        pltpu.sync_copy(out_v, out_hbm.at[pl.ds(rbase, rows_per_worker)])

    return sc_rank


@jax.jit
def kernel(x, comb):
    batch = x.shape[0]
    x32 = x.astype(jnp.int32)
    comb32 = comb.astype(jnp.int32)
    out32 = _make_sc_call(batch)(x32, comb32)
    return out32.astype(jnp.int64)


# speedup vs baseline: 1.0899x; 1.0899x over previous
import functools

import jax
import jax.numpy as jnp
from jax import lax
from jax.experimental import pallas as pl
from jax.experimental.pallas import tpu as pltpu
from jax.experimental.pallas import tpu_sc as plsc

_M = 32
_NCOLS = 33
_LANES = 16
_NUM_CORES = 2
_NUM_SUBCORES = 16
_NUM_WORKERS = _NUM_CORES * _NUM_SUBCORES


def _make_sc_call(batch):
    rows_per_worker = batch // _NUM_WORKERS
    groups = rows_per_worker // _LANES
    mesh = plsc.VectorSubcoreMesh(
        core_axis_name="c", subcore_axis_name="s",
        num_cores=_NUM_CORES, num_subcores=_NUM_SUBCORES)

    @functools.partial(
        pl.kernel,
        mesh=mesh,
        out_type=jax.ShapeDtypeStruct((batch * 2,), jnp.int32),
        scratch_types=[
            pltpu.VMEM((rows_per_worker, 2 * _M), jnp.int32),
            pltpu.VMEM((_NCOLS, 2 * _NCOLS), jnp.int32),
            pltpu.VMEM((2 * rows_per_worker,), jnp.int32),
        ],
        compiler_params=pltpu.CompilerParams(
            needs_layout_passes=False,
            disable_bounds_checks=True,
            disable_semaphore_checks=True,
        ),
    )
    def sc_rank(x_hbm, comb_hbm, out_hbm, x_v, comb_v, out_v):
        wid = (lax.axis_index("s") * jnp.int32(_NUM_CORES)
               + lax.axis_index("c"))
        rbase = wid * jnp.int32(rows_per_worker)
        pltpu.sync_copy(comb_hbm, comb_v)
        pltpu.sync_copy(x_hbm.at[pl.ds(rbase, rows_per_worker)], x_v)
        lane = lax.iota(jnp.int32, _LANES)
        zeros = jnp.zeros((_LANES,), jnp.int32)

        def group_body(g, carry):
            rows = g * jnp.int32(_LANES) + lane
            presum2 = jnp.zeros((_LANES,), jnp.int32)
            acc = jnp.zeros((_LANES,), jnp.int32)
            for i in range(_M):
                col_i = jnp.full((_LANES,), 2 * i, jnp.int32)
                xi = plsc.load_gather(x_v, [rows, col_i])
                left2 = jnp.full((_LANES,), 2 * _M, jnp.int32) - presum2
                row_i = jnp.full((_LANES,), _M - 1 - i, jnp.int32)
                cval = plsc.load_gather(comb_v, [row_i, left2])
                acc = acc + cval * xi
                presum2 = presum2 + xi + xi
            rows2 = rows + rows
            plsc.store_scatter(out_v, [rows2], acc)
            plsc.store_scatter(out_v, [rows2 + jnp.int32(1)], zeros)
            return carry

        lax.fori_loop(jnp.int32(0), jnp.int32(groups), group_body,
                      jnp.int32(0))
        pltpu.sync_copy(
            out_v, out_hbm.at[pl.ds(rbase * jnp.int32(2),
                                    2 * rows_per_worker)])

    return sc_rank


@jax.jit
def kernel(x, comb):
    batch = x.shape[0]
    x_flat = lax.bitcast_convert_type(x, jnp.int32).reshape(batch, 2 * _M)
    comb_flat = lax.bitcast_convert_type(comb, jnp.int32).reshape(
        _NCOLS, 2 * _NCOLS)
    out_flat = _make_sc_call(batch)(x_flat, comb_flat)
    out_pairs = out_flat.reshape(batch, 2)
    return lax.bitcast_convert_type(out_pairs, jnp.int64)


# speedup vs baseline: 4.0767x; 3.7405x over previous
import functools

import jax
import jax.numpy as jnp
from jax import lax
from jax.experimental import pallas as pl
from jax.experimental.pallas import tpu as pltpu
from jax.experimental.pallas import tpu_sc as plsc

_M = 32
_NCOLS = 33
_LANES = 16
_NUM_CORES = 2
_NUM_SUBCORES = 16
_NUM_WORKERS = _NUM_CORES * _NUM_SUBCORES


def _make_sc_call(batch):
    rows_per_worker = batch // _NUM_WORKERS
    groups = rows_per_worker // _LANES
    mesh = plsc.VectorSubcoreMesh(
        core_axis_name="c", subcore_axis_name="s",
        num_cores=_NUM_CORES, num_subcores=_NUM_SUBCORES)

    @functools.partial(
        pl.kernel,
        mesh=mesh,
        out_type=jax.ShapeDtypeStruct((batch,), jnp.int32),
        scratch_types=[
            pltpu.VMEM((_M, rows_per_worker), jnp.int32),
            pltpu.VMEM((_NCOLS, _NCOLS), jnp.int32),
            pltpu.VMEM((rows_per_worker,), jnp.int32),
        ],
        compiler_params=pltpu.CompilerParams(
            needs_layout_passes=False,
            disable_bounds_checks=True,
            disable_semaphore_checks=True,
        ),
    )
    def sc_rank(xt_hbm, comb_hbm, out_hbm, x_v, comb_v, out_v):
        wid = (lax.axis_index("s") * jnp.int32(_NUM_CORES)
               + lax.axis_index("c"))
        rbase = wid * jnp.int32(rows_per_worker)
        pltpu.sync_copy(comb_hbm, comb_v)
        pltpu.sync_copy(xt_hbm.at[:, pl.ds(rbase, rows_per_worker)], x_v)

        def group_body(g, carry):
            gbase = g * jnp.int32(_LANES)
            presum = jnp.zeros((_LANES,), jnp.int32)
            acc = jnp.zeros((_LANES,), jnp.int32)
            for i in range(_M):
                xi = x_v[i, pl.ds(gbase, _LANES)]
                left = jnp.full((_LANES,), _M, jnp.int32) - presum
                row_i = jnp.full((_LANES,), _M - 1 - i, jnp.int32)
                cval = plsc.load_gather(comb_v, [row_i, left])
                acc = acc + cval * xi
                presum = presum + xi
            out_v[pl.ds(gbase, _LANES)] = acc
            return carry

        lax.fori_loop(jnp.int32(0), jnp.int32(groups), group_body,
                      jnp.int32(0))
        pltpu.sync_copy(out_v, out_hbm.at[pl.ds(rbase, rows_per_worker)])

    return sc_rank


@jax.jit
def kernel(x, comb):
    batch = x.shape[0]
    xt32 = x.T.astype(jnp.int32)
    comb32 = comb.astype(jnp.int32)
    out32 = _make_sc_call(batch)(xt32, comb32)
    return out32.astype(jnp.int64)
